# baseline (speedup 1.0000x reference)
import jax
import jax.numpy as jnp
from jax import lax
from jax.experimental import pallas as pl
from jax.experimental.pallas import tpu as pltpu
from jax.experimental.pallas import tpu_sc as plsc

N = 10000
NPAD = 10240
C = 128
E = 320000
NC = 2
NS = 16
NW = NC * NS
EPW = 10240
K = 64
NCH = EPW // K
RPT = NPAD // NS
NRCH = RPT // K
L = 16
NB = 3
IMAX = 2147483647


def _sc_body(rows_hbm, cols_hbm, wt_hbm, parts_hbm,
             row_v, col_v,
             ridx0_v, cidx0_v, ridx1_v, cidx1_v, ridx2_v, cidx2_v,
             grow0_v, grow1_v, grow2_v,
             minvec_v, minsall_v, mins_sh, acc_sh,
             g0sem, g1sem, g2sem, s0sem, s1sem, s2sem):
    c = lax.axis_index("c")
    s = lax.axis_index("s")
    wid = c * NS + s
    mirror = (1 - c) * NS + s

    pltpu.sync_copy(rows_hbm.at[pl.ds(wid * EPW, EPW)], row_v)
    pltpu.sync_copy(rows_hbm.at[pl.ds(mirror * EPW, EPW)], col_v)

    def minbody(i, mv):
        for u in range(4):
            a = row_v[pl.ds((4 * i + u) * L, L)]
            b2 = col_v[pl.ds((4 * i + u) * L, L)]
            mv = jnp.minimum(mv, jnp.minimum(a, b2))
        return mv

    mv = lax.fori_loop(0, EPW // (4 * L), minbody,
                       jnp.full((L,), IMAX, jnp.int32))
    minvec_v[...] = mv
    pltpu.sync_copy(minvec_v, mins_sh.at[s])
    colcp = pltpu.async_copy(cols_hbm.at[pl.ds(wid * EPW, EPW)], col_v, g1sem)

    for j in range(K // L):
        cidx0_v[pl.ds(j * L, L)] = lax.iota(jnp.int32, L) + (N + j * L)
    pltpu.async_copy(wt_hbm.at[cidx0_v], grow0_v, g0sem).wait()
    r0 = s * RPT
    for k in range(NRCH):
        pltpu.async_copy(grow0_v, acc_sh.at[pl.ds(r0 + k * K, K)], s0sem)
    for k in range(NRCH):
        pltpu.make_async_copy(
            grow0_v, acc_sh.at[pl.ds(r0 + k * K, K)], s0sem).wait()

    plsc.subcore_barrier()

    pltpu.sync_copy(mins_sh, minsall_v)
    mv2 = minsall_v[0]
    for t in range(1, NS):
        mv2 = jnp.minimum(mv2, minsall_v[t])
    m = mv2[0]
    for t in range(1, L):
        m = jnp.minimum(m, mv2[t])

    colcp.wait()

    bufs = [(cidx0_v, ridx0_v, grow0_v, g0sem, s0sem),
            (cidx1_v, ridx1_v, grow1_v, g1sem, s1sem),
            (cidx2_v, ridx2_v, grow2_v, g2sem, s2sem)]

    def stage(b, cc):
        cidx, ridx = bufs[b][0], bufs[b][1]
        base = cc * K
        for j in range(K // L):
            off = j * L
            ridx[pl.ds(off, L)] = row_v[pl.ds(base + off, L)] - m
            cidx[pl.ds(off, L)] = col_v[pl.ds(base + off, L)]

    def gstart(b):
        cidx, _, grow, gsem, _ = bufs[b]
        pltpu.async_copy(wt_hbm.at[cidx], grow, gsem)

    def gwait(b):
        cidx, _, grow, gsem, _ = bufs[b]
        pltpu.make_async_copy(wt_hbm.at[cidx], grow, gsem).wait()

    def sstart(b):
        _, ridx, grow, _, ssem = bufs[b]
        pltpu.async_copy(grow, acc_sh.at[ridx], ssem, add=True)

    def swait(b):
        _, ridx, grow, _, ssem = bufs[b]
        pltpu.make_async_copy(grow, acc_sh.at[ridx], ssem).wait()

    stage(0, 0)
    gstart(0)
    stage(1, 1)
    gstart(1)
    stage(2, 2)
    gstart(2)
    gwait(0)
    sstart(0)
    gwait(1)
    sstart(1)
    swait(0)
    stage(0, 3)
    gstart(0)

    def triple_body(g, _):
        for k in range(NB):
            cc = NB * g + 4 + k
            b = (1 + k) % NB
            gwait((b + 1) % NB)
            sstart((b + 1) % NB)
            swait(b)
            stage(b, cc)
            gstart(b)
        return 0

    lax.fori_loop(0, (NCH - 4) // NB, triple_body, 0)

    gwait(2)
    sstart(2)
    gwait(0)
    sstart(0)
    swait(1)
    swait(2)
    swait(0)

    plsc.subcore_barrier()

    wbufs = [(grow0_v, g0sem, s0sem), (grow1_v, g1sem, s1sem)]

    def rd(k):
        gb, gs, _ = wbufs[k % 2]
        pltpu.async_copy(acc_sh.at[pl.ds(r0 + k * K, K)], gb, gs)

    def rdwait(k):
        gb, gs, _ = wbufs[k % 2]
        pltpu.make_async_copy(acc_sh.at[pl.ds(r0 + k * K, K)], gb, gs).wait()

    def wr(k):
        gb, _, ss = wbufs[k % 2]
        pltpu.async_copy(gb, parts_hbm.at[c, pl.ds(r0 + k * K, K)], ss)

    def wrwait(k):
        gb, _, ss = wbufs[k % 2]
        pltpu.make_async_copy(
            gb, parts_hbm.at[c, pl.ds(r0 + k * K, K)], ss).wait()

    rd(0)
    for k in range(NRCH):
        if k >= 1:
            wrwait(k - 1)
        if k + 1 < NRCH:
            rd(k + 1)
        rdwait(k)
        wr(k)
    wrwait(NRCH - 1)


_sc_call = pl.kernel(
    _sc_body,
    out_type=jax.ShapeDtypeStruct((NC, NPAD, C), jnp.float32),
    mesh=plsc.VectorSubcoreMesh(core_axis_name="c", subcore_axis_name="s"),
    scratch_types=[
        pltpu.VMEM((EPW,), jnp.int32),
        pltpu.VMEM((EPW,), jnp.int32),
        pltpu.VMEM((K,), jnp.int32),
        pltpu.VMEM((K,), jnp.int32),
        pltpu.VMEM((K,), jnp.int32),
        pltpu.VMEM((K,), jnp.int32),
        pltpu.VMEM((K,), jnp.int32),
        pltpu.VMEM((K,), jnp.int32),
        pltpu.VMEM((K, C), jnp.float32),
        pltpu.VMEM((K, C), jnp.float32),
        pltpu.VMEM((K, C), jnp.float32),
        pltpu.VMEM((L,), jnp.int32),
        pltpu.VMEM((NS, L), jnp.int32),
        pltpu.VMEM_SHARED((NS, L), jnp.int32),
        pltpu.VMEM_SHARED((NPAD, C), jnp.float32),
        pltpu.SemaphoreType.DMA,
        pltpu.SemaphoreType.DMA,
        pltpu.SemaphoreType.DMA,
        pltpu.SemaphoreType.DMA,
        pltpu.SemaphoreType.DMA,
        pltpu.SemaphoreType.DMA,
    ],
)


def _merge_body(p_ref, b_ref, o_ref):
    o_ref[...] = p_ref[0] + p_ref[1] + b_ref[...]


def _merge(parts, b):
    rb = 2000
    return pl.pallas_call(
        _merge_body,
        grid=(N // rb,),
        in_specs=[
            pl.BlockSpec((NC, rb, C), lambda i: (0, i, 0)),
            pl.BlockSpec((1, C), lambda i: (0, 0)),
        ],
        out_specs=pl.BlockSpec((rb, C), lambda i: (i, 0)),
        out_shape=jax.ShapeDtypeStruct((N, C), jnp.float32),
    )(parts, b.reshape(1, C))


@jax.jit
def _impl(edge_index, W, b):
    row = edge_index[0].astype(jnp.int32).reshape(NW, E // NW)
    col = edge_index[1].astype(jnp.int32).reshape(NW, E // NW)
    npe = EPW - E // NW
    row = jnp.concatenate([row, row[:, :npe]], axis=1).reshape(-1)
    col = jnp.concatenate(
        [col, jnp.broadcast_to(jnp.arange(N, N + npe, dtype=jnp.int32),
                               (NW, npe))], axis=1).reshape(-1)
    wt = jnp.concatenate(
        [W.T.reshape(N, C), jnp.zeros((npe, C), jnp.float32)], axis=0)
    parts = _sc_call(row, col, wt)
    return _merge(parts, b)


def kernel(edge_index, W, b):
    return _impl(edge_index, W, b)


# speedup vs baseline: 1.0541x; 1.0541x over previous
import jax
import jax.numpy as jnp
from jax import lax
from jax.experimental import pallas as pl
from jax.experimental.pallas import tpu as pltpu
from jax.experimental.pallas import tpu_sc as plsc

N = 10000
NPAD = 10240
C = 128
E = 320000
NC = 2
NS = 16
NW = NC * NS
EPW = 10240
K = 64
NCH = EPW // K
RPT = NPAD // NS
NRCH = RPT // K
L = 16
NB = 3
IMAX = 2147483647


def _sc_body(rows_hbm, cols_hbm, wt_hbm, parts_hbm,
             row_v, col_v,
             ridx0_v, cidx0_v, ridx1_v, cidx1_v, ridx2_v, cidx2_v,
             grow0_v, grow1_v, grow2_v,
             minvec_v, minsall_v, mins_sh, acc_sh,
             g0sem, g1sem, g2sem, s0sem, s1sem, s2sem):
    c = lax.axis_index("c")
    s = lax.axis_index("s")
    wid = c * NS + s
    mirror = (1 - c) * NS + s

    pltpu.sync_copy(rows_hbm.at[pl.ds(wid * EPW, EPW)], row_v)
    pltpu.sync_copy(rows_hbm.at[pl.ds(mirror * EPW, EPW)], col_v)

    def minbody(i, mv):
        for u in range(4):
            a = row_v[pl.ds((4 * i + u) * L, L)]
            b2 = col_v[pl.ds((4 * i + u) * L, L)]
            mv = jnp.minimum(mv, jnp.minimum(a, b2))
        return mv

    mv = lax.fori_loop(0, EPW // (4 * L), minbody,
                       jnp.full((L,), IMAX, jnp.int32))
    minvec_v[...] = mv
    pltpu.sync_copy(minvec_v, mins_sh.at[s])
    colcp = pltpu.async_copy(cols_hbm.at[pl.ds(wid * EPW, EPW)], col_v, g1sem)

    for j in range(K // L):
        cidx0_v[pl.ds(j * L, L)] = lax.iota(jnp.int32, L) + (N + j * L)
    pltpu.async_copy(wt_hbm.at[cidx0_v], grow0_v, g0sem).wait()
    r0 = s * RPT
    for k in range(NRCH):
        pltpu.async_copy(grow0_v, acc_sh.at[pl.ds(r0 + k * K, K)], s0sem)
    for k in range(NRCH):
        pltpu.make_async_copy(
            grow0_v, acc_sh.at[pl.ds(r0 + k * K, K)], s0sem).wait()

    plsc.subcore_barrier()

    pltpu.sync_copy(mins_sh, minsall_v)
    mv2 = minsall_v[0]
    for t in range(1, NS):
        mv2 = jnp.minimum(mv2, minsall_v[t])
    m = mv2[0]
    for t in range(1, L):
        m = jnp.minimum(m, mv2[t])

    colcp.wait()

    bufs = [(cidx0_v, ridx0_v, grow0_v, g0sem, s0sem),
            (cidx1_v, ridx1_v, grow1_v, g1sem, s1sem),
            (cidx2_v, ridx2_v, grow2_v, g2sem, s2sem)]

    def stage(b, cc):
        cidx, ridx = bufs[b][0], bufs[b][1]
        base = cc * K
        for j in range(K // L):
            off = j * L
            ridx[pl.ds(off, L)] = row_v[pl.ds(base + off, L)] - m
            cidx[pl.ds(off, L)] = col_v[pl.ds(base + off, L)]

    def gstart(b):
        cidx, _, grow, gsem, _ = bufs[b]
        pltpu.async_copy(wt_hbm.at[cidx], grow, gsem)

    def gwait(b):
        cidx, _, grow, gsem, _ = bufs[b]
        pltpu.make_async_copy(wt_hbm.at[cidx], grow, gsem).wait()

    def sstart(b):
        _, ridx, grow, _, ssem = bufs[b]
        pltpu.async_copy(grow, acc_sh.at[ridx], ssem, add=True)

    def swait(b):
        _, ridx, grow, _, ssem = bufs[b]
        pltpu.make_async_copy(grow, acc_sh.at[ridx], ssem).wait()

    stage(0, 0)
    gstart(0)
    stage(1, 1)
    gstart(1)
    stage(2, 2)
    gstart(2)
    gwait(0)
    sstart(0)
    swait(0)
    stage(0, 3)
    gstart(0)
    gwait(1)
    sstart(1)

    def triple_body(g, _):
        for k in range(NB):
            cc = NB * g + 4 + k
            b = (1 + k) % NB
            swait(b)
            stage(b, cc)
            gstart(b)
            gwait((b + 1) % NB)
            sstart((b + 1) % NB)
        return 0

    lax.fori_loop(0, (NCH - 4) // NB, triple_body, 0)

    gwait(2)
    sstart(2)
    gwait(0)
    sstart(0)
    swait(1)
    swait(2)
    swait(0)

    plsc.subcore_barrier()

    wbufs = [(grow0_v, g0sem, s0sem), (grow1_v, g1sem, s1sem)]

    def rd(k):
        gb, gs, _ = wbufs[k % 2]
        pltpu.async_copy(acc_sh.at[pl.ds(r0 + k * K, K)], gb, gs)

    def rdwait(k):
        gb, gs, _ = wbufs[k % 2]
        pltpu.make_async_copy(acc_sh.at[pl.ds(r0 + k * K, K)], gb, gs).wait()

    def wr(k):
        gb, _, ss = wbufs[k % 2]
        pltpu.async_copy(gb, parts_hbm.at[c, pl.ds(r0 + k * K, K)], ss)

    def wrwait(k):
        gb, _, ss = wbufs[k % 2]
        pltpu.make_async_copy(
            gb, parts_hbm.at[c, pl.ds(r0 + k * K, K)], ss).wait()

    rd(0)
    for k in range(NRCH):
        if k >= 1:
            wrwait(k - 1)
        if k + 1 < NRCH:
            rd(k + 1)
        rdwait(k)
        wr(k)
    wrwait(NRCH - 1)


_sc_call = pl.kernel(
    _sc_body,
    out_type=jax.ShapeDtypeStruct((NC, NPAD, C), jnp.float32),
    mesh=plsc.VectorSubcoreMesh(core_axis_name="c", subcore_axis_name="s"),
    scratch_types=[
        pltpu.VMEM((EPW,), jnp.int32),
        pltpu.VMEM((EPW,), jnp.int32),
        pltpu.VMEM((K,), jnp.int32),
        pltpu.VMEM((K,), jnp.int32),
        pltpu.VMEM((K,), jnp.int32),
        pltpu.VMEM((K,), jnp.int32),
        pltpu.VMEM((K,), jnp.int32),
        pltpu.VMEM((K,), jnp.int32),
        pltpu.VMEM((K, C), jnp.float32),
        pltpu.VMEM((K, C), jnp.float32),
        pltpu.VMEM((K, C), jnp.float32),
        pltpu.VMEM((L,), jnp.int32),
        pltpu.VMEM((NS, L), jnp.int32),
        pltpu.VMEM_SHARED((NS, L), jnp.int32),
        pltpu.VMEM_SHARED((NPAD, C), jnp.float32),
        pltpu.SemaphoreType.DMA,
        pltpu.SemaphoreType.DMA,
        pltpu.SemaphoreType.DMA,
        pltpu.SemaphoreType.DMA,
        pltpu.SemaphoreType.DMA,
        pltpu.SemaphoreType.DMA,
    ],
)


def _merge_body(p_ref, b_ref, o_ref):
    o_ref[...] = p_ref[0] + p_ref[1] + b_ref[...]


def _merge(parts, b):
    rb = 2000
    return pl.pallas_call(
        _merge_body,
        grid=(N // rb,),
        in_specs=[
            pl.BlockSpec((NC, rb, C), lambda i: (0, i, 0)),
            pl.BlockSpec((1, C), lambda i: (0, 0)),
        ],
        out_specs=pl.BlockSpec((rb, C), lambda i: (i, 0)),
        out_shape=jax.ShapeDtypeStruct((N, C), jnp.float32),
    )(parts, b.reshape(1, C))


@jax.jit
def _impl(edge_index, W, b):
    row = edge_index[0].astype(jnp.int32).reshape(NW, E // NW)
    col = edge_index[1].astype(jnp.int32).reshape(NW, E // NW)
    npe = EPW - E // NW
    row = jnp.concatenate([row, row[:, :npe]], axis=1).reshape(-1)
    col = jnp.concatenate(
        [col, jnp.broadcast_to(jnp.arange(N, N + npe, dtype=jnp.int32),
                               (NW, npe))], axis=1).reshape(-1)
    wt = jnp.concatenate(
        [W.T.reshape(N, C), jnp.zeros((npe, C), jnp.float32)], axis=0)
    parts = _sc_call(row, col, wt)
    return _merge(parts, b)


def kernel(edge_index, W, b):
    return _impl(edge_index, W, b)
